# baseline (speedup 1.0000x reference)
import functools

import jax
import jax.numpy as jnp
import numpy as np
from jax import lax
from jax.experimental import pallas as pl
from jax.experimental.pallas import tpu as pltpu
from jax.experimental.pallas import tpu_sc as plsc

N = 10000
E = 160000
NODE = 128
NUM_IRR = 224
SPH = 480
NB = 20
HID = NODE + NUM_IRR * 2

_reps = np.array([1] * 128 + [3] * 64 + [5] * 32)
_col_of = np.repeat(np.arange(NUM_IRR), _reps)
_SEL_TAIL = np.zeros((96, 352), dtype=np.float32)
for _j, _c in enumerate(_col_of[128:]):
    _SEL_TAIL[_c - 128, _j] = 1.0
_SEL2 = np.zeros((192, 704), dtype=np.float32)
_SEL2[:96, :352] = _SEL_TAIL
_SEL2[96:, 352:] = _SEL_TAIL

BN = 1000
BE = 1280
HIDP = 640
SPHP = 512


def _mlp_body(x_ref, xsph_ref, w1_ref, b1_ref, w2_ref, b2_ref, o_ref):
    x = x_ref[...]
    h = jnp.dot(x, w1_ref[...], preferred_element_type=jnp.float32,
                precision=jax.lax.Precision.HIGHEST) + b1_ref[...]
    h = h * jax.nn.sigmoid(h)
    so = jnp.dot(h, w2_ref[...], preferred_element_type=jnp.float32,
                 precision=jax.lax.Precision.HIGHEST) + b2_ref[...]
    pad = jnp.zeros((so.shape[0], HIDP - SPH), jnp.float32)
    hi = jnp.concatenate([xsph_ref[...], pad], axis=1)

    def rnd(u):
        return (u + 0x7FFF + ((u >> 16) & 1)) >> 16

    ulo = rnd(jax.lax.bitcast_convert_type(so, jnp.uint32))
    uhi = rnd(jax.lax.bitcast_convert_type(hi, jnp.uint32))
    o_ref[...] = jax.lax.bitcast_convert_type(ulo | (uhi << 16), jnp.float32)


def _edge_body(gtab_ref, rbf_ref, fcut_ref, rsh_ref,
               wr_ref, br_ref, sel_ref, msg_ref):
    fw = jnp.dot(rbf_ref[...].astype(jnp.bfloat16), wr_ref[...],
                 preferred_element_type=jnp.float32) + br_ref[...]
    fw = fw * fcut_ref[...]
    u = jax.lax.bitcast_convert_type(gtab_ref[...], jnp.uint32)
    gso = jax.lax.bitcast_convert_type(u << 16, jnp.float32)
    gsph = jax.lax.bitcast_convert_type(u & jnp.uint32(0xFFFF0000), jnp.float32)
    fo = gso * fw
    ms = fo[:, 2 * NUM_IRR:HID]
    gtails = jnp.concatenate([fo[:, 128:NUM_IRR], fo[:, NUM_IRR + 128:2 * NUM_IRR]],
                             axis=1).astype(jnp.bfloat16)
    tails = jnp.dot(gtails, sel_ref[...], preferred_element_type=jnp.float32)
    rep_gs = jnp.concatenate([fo[:, :128], tails[:, :352]], axis=1)
    rep_ge = jnp.concatenate([fo[:, NUM_IRR:NUM_IRR + 128], tails[:, 352:]],
                             axis=1)
    msph = gsph[:, :SPH] * rep_gs + rsh_ref[...] * rep_ge
    pad = jnp.zeros((msph.shape[0], HIDP - NODE - SPH), jnp.float32)
    msg_ref[...] = jnp.concatenate([ms, msph, pad], axis=1)


TBW = HIDP


def _mlp(x_scalar, x_spherical, W1, b1, W2, b2):
    return pl.pallas_call(
        _mlp_body,
        grid=(N // BN,),
        in_specs=[
            pl.BlockSpec((BN, NODE), lambda i: (i, 0)),
            pl.BlockSpec((BN, SPH), lambda i: (i, 0)),
            pl.BlockSpec((NODE, NODE), lambda i: (0, 0)),
            pl.BlockSpec((NODE,), lambda i: (0,)),
            pl.BlockSpec((NODE, HIDP), lambda i: (0, 0)),
            pl.BlockSpec((HIDP,), lambda i: (0,)),
        ],
        out_specs=pl.BlockSpec((BN, TBW), lambda i: (i, 0)),
        out_shape=jax.ShapeDtypeStruct((N, TBW), jnp.float32),
    )(x_scalar, x_spherical, W1, b1, W2, b2)


def _edge_math(g_tab, rbf, fcut, rsh, Wr, br, sel, ebase, ecount):
    off = ebase // BE
    return pl.pallas_call(
        _edge_body,
        grid=(ecount // BE,),
        in_specs=[
            pl.BlockSpec((BE, TBW), lambda i: (i, 0)),
            pl.BlockSpec((BE, NB), lambda i: (i + off, 0)),
            pl.BlockSpec((BE, 1), lambda i: (i + off, 0)),
            pl.BlockSpec((BE, SPH), lambda i: (i + off, 0)),
            pl.BlockSpec((NB, HIDP), lambda i: (0, 0)),
            pl.BlockSpec((HIDP,), lambda i: (0,)),
            pl.BlockSpec((192, 704), lambda i: (0, 0)),
        ],
        out_specs=pl.BlockSpec((BE, HIDP), lambda i: (i, 0)),
        out_shape=jax.ShapeDtypeStruct((ecount, HIDP), jnp.float32),
    )(g_tab, rbf, fcut, rsh, Wr, br, sel)


def _combine_body(xs_ref, xsph_ref, parts_ref, ns_ref, nsph_ref):
    p = parts_ref[...]
    q = p[:, 0] + p[:, 1]
    ns_ref[...] = xs_ref[...] + q[0]
    sph = jnp.concatenate([q[1], q[2], q[3], q[4]], axis=1)[:, :SPH]
    nsph_ref[...] = xsph_ref[...] + sph


def _combine(x_scalar, x_spherical, parts):
    return pl.pallas_call(
        _combine_body,
        grid=(N // BN,),
        in_specs=[
            pl.BlockSpec((BN, NODE), lambda i: (i, 0)),
            pl.BlockSpec((BN, SPH), lambda i: (i, 0)),
            pl.BlockSpec((_NPASS, _NC, BN, 128), lambda i: (0, 0, i, 0)),
        ],
        out_specs=[
            pl.BlockSpec((BN, NODE), lambda i: (i, 0)),
            pl.BlockSpec((BN, SPH), lambda i: (i, 0)),
        ],
        out_shape=[
            jax.ShapeDtypeStruct((N, NODE), jnp.float32),
            jax.ShapeDtypeStruct((N, SPH), jnp.float32),
        ],
    )(x_scalar, x_spherical, parts)


_NC, _NS = 2, 16
_NW = _NC * _NS
_EPW = E // _NW
_GCH = 40

def _sc_mesh():
    return plsc.VectorSubcoreMesh(core_axis_name="c", subcore_axis_name="s")


def _gather_body(epw, ebase, tab_hbm, dst_hbm, out_tab,
                 idx_all, buf_a, buf_b,
                 gsem_a, gsem_b, wsem_a, wsem_b):
    ngch = epw // _GCH
    wid = lax.axis_index("s") * _NC + lax.axis_index("c")
    base = wid * epw
    pltpu.sync_copy(dst_hbm.at[pl.ds(ebase + base, epw)], idx_all.at[pl.ds(0, epw)])

    def gstart(chunk, buf, gsem):
        off = pl.multiple_of(chunk * _GCH, 8)
        pltpu.async_copy(tab_hbm.at[idx_all.at[pl.ds(off, _GCH)]], buf, gsem)

    def gwait(buf, gsem):
        pltpu.make_async_copy(tab_hbm.at[pl.ds(0, _GCH)], buf, gsem).wait()

    def wstart(chunk, buf, wsem):
        cb = pl.multiple_of(base + chunk * _GCH, 8)
        pltpu.async_copy(buf, out_tab.at[pl.ds(cb, _GCH)], wsem)

    def wwait(buf, wsem):
        pltpu.make_async_copy(buf, out_tab.at[pl.ds(0, _GCH)], wsem).wait()

    gstart(0, buf_a, gsem_a)
    gstart(1, buf_b, gsem_b)

    def pair(j, carry):
        gwait(buf_a, gsem_a)
        wstart(2 * j, buf_a, wsem_a)

        @pl.when(2 * j + 1 < ngch)
        def _():
            gwait(buf_b, gsem_b)
            wstart(2 * j + 1, buf_b, wsem_b)

        @pl.when(2 * j + 2 < ngch)
        def _():
            wwait(buf_a, wsem_a)
            gstart(2 * j + 2, buf_a, gsem_a)

        @pl.when(2 * j + 3 < ngch)
        def _():
            wwait(buf_b, wsem_b)
            gstart(2 * j + 3, buf_b, gsem_b)

        return carry

    lax.fori_loop(0, (ngch + 1) // 2, pair, 0)
    wwait(buf_a, wsem_a)
    wwait(buf_b, wsem_b)


def _sc_gather(table, dst, ebase, ecount):
    return pl.kernel(
        functools.partial(_gather_body, ecount // _NW, ebase),
        out_type=jax.ShapeDtypeStruct((ecount, TBW), jnp.float32),
        mesh=_sc_mesh(),
        scratch_types=[
            pltpu.VMEM((_EPW,), jnp.int32),
            pltpu.VMEM((_GCH, TBW), jnp.float32),
            pltpu.VMEM((_GCH, TBW), jnp.float32),
            pltpu.SemaphoreType.DMA,
            pltpu.SemaphoreType.DMA,
            pltpu.SemaphoreType.DMA,
            pltpu.SemaphoreType.DMA,
        ],
    )(table, dst)


_SCH = 128
_NPASS = HIDP // 128
_EC0 = 79360
_EC1 = E - _EC0
_EPT0 = _EC0 // _NW
_EPT1 = _EC1 // _NW
_REM0 = _EPT0 - (_EPT0 // _SCH) * _SCH
_REM1 = _EPT1 - (_EPT1 // _SCH) * _SCH


def _scatter_body(msg0_hbm, msg1_hbm, src_hbm, zeros_hbm, out_hbm,
                  idx_a, idx_b, idx_r0, idx_r1, msg_a, msg_b, msg_r0, msg_r1,
                  acc_sh, sem_a, sem_b):
    c = lax.axis_index("c")
    s = lax.axis_index("s")

    def chunk_loop(msg_hbm, ebase, ec, ept, rem, idx_r, msg_r, col):
        nfull = ept // _SCH
        lbase = c * (ec // 2) + s * ept
        gbase = ebase + lbase

        def start(j, idx_v, msg_v, sem):
            lo = pl.multiple_of(lbase + j * _SCH, 8)
            go = pl.multiple_of(gbase + j * _SCH, 8)
            pltpu.async_copy(src_hbm.at[pl.ds(go, _SCH)], idx_v, sem)
            pltpu.async_copy(msg_hbm.at[pl.ds(lo, _SCH), pl.ds(col, 128)],
                             msg_v, sem)

        def wait_and_scatter(idx_v, msg_v, sem):
            pltpu.make_async_copy(src_hbm.at[pl.ds(0, _SCH)], idx_v, sem).wait()
            pltpu.make_async_copy(msg_hbm.at[pl.ds(0, _SCH), pl.ds(col, 128)],
                                  msg_v, sem).wait()
            pltpu.sync_copy(msg_v, acc_sh.at[idx_v], add=True)

        start(0, idx_a, msg_a, sem_a)

        def pair(j, carry):
            @pl.when(2 * j + 1 < nfull)
            def _():
                start(2 * j + 1, idx_b, msg_b, sem_b)

            wait_and_scatter(idx_a, msg_a, sem_a)

            @pl.when(2 * j + 2 < nfull)
            def _():
                start(2 * j + 2, idx_a, msg_a, sem_a)

            @pl.when(2 * j + 1 < nfull)
            def _():
                wait_and_scatter(idx_b, msg_b, sem_b)

            return carry

        lax.fori_loop(0, (nfull + 1) // 2, pair, 0)

        lo = pl.multiple_of(lbase + nfull * _SCH, 8)
        go = pl.multiple_of(gbase + nfull * _SCH, 8)
        pltpu.sync_copy(src_hbm.at[pl.ds(go, rem)], idx_r)
        pltpu.sync_copy(msg_hbm.at[pl.ds(lo, rem), pl.ds(col, 128)], msg_r)
        pltpu.sync_copy(msg_r, acc_sh.at[idx_r], add=True)

    def one_pass(p, carry):
        col = pl.multiple_of(p * 128, 128)
        @pl.when(s < _NS - 1)
        def _():
            pltpu.sync_copy(zeros_hbm.at[pl.ds(0, 624)],
                            acc_sh.at[pl.ds(s * 624, 624)])

        @pl.when(s == _NS - 1)
        def _():
            pltpu.sync_copy(zeros_hbm.at[pl.ds(0, 640)],
                            acc_sh.at[pl.ds(9360, 640)])

        plsc.subcore_barrier()
        chunk_loop(msg0_hbm, 0, _EC0, _EPT0, _REM0, idx_r0, msg_r0, col)
        chunk_loop(msg1_hbm, _EC0, _EC1, _EPT1, _REM1, idx_r1, msg_r1, col)
        plsc.subcore_barrier()

        @pl.when(s < _NS - 1)
        def _():
            pltpu.sync_copy(acc_sh.at[pl.ds(s * 624, 624)],
                            out_hbm.at[p, c].at[pl.ds(s * 624, 624)])

        @pl.when(s == _NS - 1)
        def _():
            pltpu.sync_copy(acc_sh.at[pl.ds(9360, 640)],
                            out_hbm.at[p, c].at[pl.ds(9360, 640)])

        plsc.subcore_barrier()
        return carry

    lax.fori_loop(0, _NPASS, one_pass, 0)


def _sc_scatter(msg0, msg1, src, zeros):
    return pl.kernel(
        _scatter_body,
        out_type=jax.ShapeDtypeStruct((_NPASS, _NC, N, 128), jnp.float32),
        mesh=_sc_mesh(),
        scratch_types=[
            pltpu.VMEM((_SCH,), jnp.int32),
            pltpu.VMEM((_SCH,), jnp.int32),
            pltpu.VMEM((_REM0,), jnp.int32),
            pltpu.VMEM((_REM1,), jnp.int32),
            pltpu.VMEM((_SCH, 128), jnp.float32),
            pltpu.VMEM((_SCH, 128), jnp.float32),
            pltpu.VMEM((_REM0, 128), jnp.float32),
            pltpu.VMEM((_REM1, 128), jnp.float32),
            pltpu.VMEM_SHARED((N, 128), jnp.float32),
            pltpu.SemaphoreType.DMA,
            pltpu.SemaphoreType.DMA,
        ],
    )(msg0, msg1, src, zeros)


def kernel(x_scalar, x_spherical, rbf, fcut, rsh, edge_index, W1, b1, W2, b2, Wr, br):
    W2p = jnp.pad(W2, ((0, 0), (0, HIDP - HID)))
    b2p = jnp.pad(b2, (0, HIDP - HID))
    Wrp = jnp.pad(Wr, ((0, 0), (0, HIDP - HID)))
    brp = jnp.pad(br, (0, HIDP - HID))
    table = _mlp(x_scalar, x_spherical, W1, b1, W2p, b2p)
    sel2 = jnp.asarray(_SEL2).astype(jnp.bfloat16)
    dst = edge_index[1]
    src = edge_index[0]
    wrb = Wrp.astype(jnp.bfloat16)
    msgs = []
    for ebase, ecount in ((0, _EC0), (_EC0, _EC1)):
        g_tab = _sc_gather(table, dst, ebase, ecount)
        msgs.append(_edge_math(g_tab, rbf, fcut, rsh, wrb, brp, sel2,
                               ebase, ecount))
    zeros = jnp.zeros((640, 128), jnp.float32)
    parts = _sc_scatter(msgs[0], msgs[1], src, zeros)
    return tuple(_combine(x_scalar, x_spherical, parts))


# speedup vs baseline: 1.0238x; 1.0238x over previous
import functools

import jax
import jax.numpy as jnp
import numpy as np
from jax import lax
from jax.experimental import pallas as pl
from jax.experimental.pallas import tpu as pltpu
from jax.experimental.pallas import tpu_sc as plsc

N = 10000
E = 160000
NODE = 128
NUM_IRR = 224
SPH = 480
NB = 20
HID = NODE + NUM_IRR * 2

_reps = np.array([1] * 128 + [3] * 64 + [5] * 32)
_col_of = np.repeat(np.arange(NUM_IRR), _reps)
_SEL_TAIL = np.zeros((96, 352), dtype=np.float32)
for _j, _c in enumerate(_col_of[128:]):
    _SEL_TAIL[_c - 128, _j] = 1.0
_SEL2 = np.zeros((192, 704), dtype=np.float32)
_SEL2[:96, :352] = _SEL_TAIL
_SEL2[96:, 352:] = _SEL_TAIL

BN = 1000
BE = 1280
HIDP = 640
SPHP = 512


def _mlp_body(x_ref, xsph_ref, w1_ref, b1_ref, w2_ref, b2_ref, o_ref):
    x = x_ref[...]
    h = jnp.dot(x, w1_ref[...], preferred_element_type=jnp.float32,
                precision=jax.lax.Precision.HIGHEST) + b1_ref[...]
    h = h * jax.nn.sigmoid(h)
    so = jnp.dot(h, w2_ref[...], preferred_element_type=jnp.float32,
                 precision=jax.lax.Precision.HIGHEST) + b2_ref[...]
    pad = jnp.zeros((so.shape[0], HIDP - SPH), jnp.float32)
    hi = jnp.concatenate([xsph_ref[...], pad], axis=1)

    def rnd(u):
        return (u + 0x7FFF + ((u >> 16) & 1)) >> 16

    ulo = rnd(jax.lax.bitcast_convert_type(so, jnp.uint32))
    uhi = rnd(jax.lax.bitcast_convert_type(hi, jnp.uint32))
    o_ref[...] = jax.lax.bitcast_convert_type(ulo | (uhi << 16), jnp.float32)


def _edge_body(gtab_ref, rbf_ref, fcut_ref, rsh_ref,
               wr_ref, br_ref, sel_ref, msg_ref):
    fw = jnp.dot(rbf_ref[...].astype(jnp.bfloat16), wr_ref[...],
                 preferred_element_type=jnp.float32) + br_ref[...]
    fw = fw * fcut_ref[...]
    u = jax.lax.bitcast_convert_type(gtab_ref[...], jnp.uint32)
    gso = jax.lax.bitcast_convert_type(u << 16, jnp.float32)
    gsph = jax.lax.bitcast_convert_type(u & jnp.uint32(0xFFFF0000), jnp.float32)
    fo = gso * fw
    ms = fo[:, 2 * NUM_IRR:HID]
    gtails = jnp.concatenate([fo[:, 128:NUM_IRR], fo[:, NUM_IRR + 128:2 * NUM_IRR]],
                             axis=1).astype(jnp.bfloat16)
    tails = jnp.dot(gtails, sel_ref[...], preferred_element_type=jnp.float32)
    rep_gs = jnp.concatenate([fo[:, :128], tails[:, :352]], axis=1)
    rep_ge = jnp.concatenate([fo[:, NUM_IRR:NUM_IRR + 128], tails[:, 352:]],
                             axis=1)
    msph = gsph[:, :SPH] * rep_gs + rsh_ref[...] * rep_ge
    pad = jnp.zeros((msph.shape[0], HIDP - NODE - SPH), jnp.float32)
    msg_ref[...] = jnp.concatenate([ms, msph, pad], axis=1)


TBW = HIDP


def _mlp(x_scalar, x_spherical, W1, b1, W2, b2):
    return pl.pallas_call(
        _mlp_body,
        grid=(N // BN,),
        in_specs=[
            pl.BlockSpec((BN, NODE), lambda i: (i, 0)),
            pl.BlockSpec((BN, SPH), lambda i: (i, 0)),
            pl.BlockSpec((NODE, NODE), lambda i: (0, 0)),
            pl.BlockSpec((NODE,), lambda i: (0,)),
            pl.BlockSpec((NODE, HIDP), lambda i: (0, 0)),
            pl.BlockSpec((HIDP,), lambda i: (0,)),
        ],
        out_specs=pl.BlockSpec((BN, TBW), lambda i: (i, 0)),
        out_shape=jax.ShapeDtypeStruct((N, TBW), jnp.float32),
    )(x_scalar, x_spherical, W1, b1, W2, b2)


def _edge_math(g_tab, rbf, fcut, rsh, Wr, br, sel, ebase, ecount):
    off = ebase // BE
    return pl.pallas_call(
        _edge_body,
        grid=(ecount // BE,),
        in_specs=[
            pl.BlockSpec((BE, TBW), lambda i: (i, 0)),
            pl.BlockSpec((BE, NB), lambda i: (i + off, 0)),
            pl.BlockSpec((BE, 1), lambda i: (i + off, 0)),
            pl.BlockSpec((BE, SPH), lambda i: (i + off, 0)),
            pl.BlockSpec((NB, HIDP), lambda i: (0, 0)),
            pl.BlockSpec((HIDP,), lambda i: (0,)),
            pl.BlockSpec((192, 704), lambda i: (0, 0)),
        ],
        out_specs=pl.BlockSpec((BE, HIDP), lambda i: (i, 0)),
        out_shape=jax.ShapeDtypeStruct((ecount, HIDP), jnp.float32),
    )(g_tab, rbf, fcut, rsh, Wr, br, sel)


def _combine_body(xs_ref, xsph_ref, parts_ref, ns_ref, nsph_ref):
    p = parts_ref[...]
    q = p[:, 0] + p[:, 1]
    ns_ref[...] = xs_ref[...] + q[0]
    sph = jnp.concatenate([q[1], q[2], q[3], q[4]], axis=1)[:, :SPH]
    nsph_ref[...] = xsph_ref[...] + sph


def _combine(x_scalar, x_spherical, parts):
    return pl.pallas_call(
        _combine_body,
        grid=(N // BN,),
        in_specs=[
            pl.BlockSpec((BN, NODE), lambda i: (i, 0)),
            pl.BlockSpec((BN, SPH), lambda i: (i, 0)),
            pl.BlockSpec((_NPASS, _NC, BN, 128), lambda i: (0, 0, i, 0)),
        ],
        out_specs=[
            pl.BlockSpec((BN, NODE), lambda i: (i, 0)),
            pl.BlockSpec((BN, SPH), lambda i: (i, 0)),
        ],
        out_shape=[
            jax.ShapeDtypeStruct((N, NODE), jnp.float32),
            jax.ShapeDtypeStruct((N, SPH), jnp.float32),
        ],
    )(x_scalar, x_spherical, parts)


_NC, _NS = 2, 16
_NW = _NC * _NS
_EPW = E // _NW
_GCH = 40

def _sc_mesh():
    return plsc.VectorSubcoreMesh(core_axis_name="c", subcore_axis_name="s")


def _gather_body(epw, ebase, tab_hbm, dst_hbm, out_tab,
                 idx_all, buf_a, buf_b,
                 gsem_a, gsem_b, wsem_a, wsem_b):
    ngch = epw // _GCH
    wid = lax.axis_index("s") * _NC + lax.axis_index("c")
    base = wid * epw
    pltpu.sync_copy(dst_hbm.at[pl.ds(ebase + base, epw)], idx_all.at[pl.ds(0, epw)])

    def gstart(chunk, buf, gsem):
        off = pl.multiple_of(chunk * _GCH, 8)
        pltpu.async_copy(tab_hbm.at[idx_all.at[pl.ds(off, _GCH)]], buf, gsem)

    def gwait(buf, gsem):
        pltpu.make_async_copy(tab_hbm.at[pl.ds(0, _GCH)], buf, gsem).wait()

    def wstart(chunk, buf, wsem):
        cb = pl.multiple_of(base + chunk * _GCH, 8)
        pltpu.async_copy(buf, out_tab.at[pl.ds(cb, _GCH)], wsem)

    def wwait(buf, wsem):
        pltpu.make_async_copy(buf, out_tab.at[pl.ds(0, _GCH)], wsem).wait()

    gstart(0, buf_a, gsem_a)
    gstart(1, buf_b, gsem_b)

    def pair(j, carry):
        gwait(buf_a, gsem_a)
        wstart(2 * j, buf_a, wsem_a)

        @pl.when(2 * j + 1 < ngch)
        def _():
            gwait(buf_b, gsem_b)
            wstart(2 * j + 1, buf_b, wsem_b)

        @pl.when(2 * j + 2 < ngch)
        def _():
            wwait(buf_a, wsem_a)
            gstart(2 * j + 2, buf_a, gsem_a)

        @pl.when(2 * j + 3 < ngch)
        def _():
            wwait(buf_b, wsem_b)
            gstart(2 * j + 3, buf_b, gsem_b)

        return carry

    lax.fori_loop(0, (ngch + 1) // 2, pair, 0)
    wwait(buf_a, wsem_a)
    wwait(buf_b, wsem_b)


def _sc_gather(table, dst, ebase, ecount):
    return pl.kernel(
        functools.partial(_gather_body, ecount // _NW, ebase),
        out_type=jax.ShapeDtypeStruct((ecount, TBW), jnp.float32),
        mesh=_sc_mesh(),
        scratch_types=[
            pltpu.VMEM((_EPW,), jnp.int32),
            pltpu.VMEM((_GCH, TBW), jnp.float32),
            pltpu.VMEM((_GCH, TBW), jnp.float32),
            pltpu.SemaphoreType.DMA,
            pltpu.SemaphoreType.DMA,
            pltpu.SemaphoreType.DMA,
            pltpu.SemaphoreType.DMA,
        ],
    )(table, dst)


_SCH = 128
_NPASS = HIDP // 128
_NFULL = _EPW // _SCH
_REM = _EPW - _NFULL * _SCH
_EPC = E // _NC


def _scatter_body(msg_hbm, src_hbm, zeros_hbm, out_hbm,
                  idx_a, idx_b, idx_r, msg_a, msg_b, msg_r,
                  acc_sh, sem_a, sem_b):
    c = lax.axis_index("c")
    s = lax.axis_index("s")
    ebase = c * _EPC + s * _EPW

    def start(j, idx_v, msg_v, sem, col):
        eb = pl.multiple_of(ebase + j * _SCH, 8)
        pltpu.async_copy(src_hbm.at[pl.ds(eb, _SCH)], idx_v, sem)
        pltpu.async_copy(msg_hbm.at[pl.ds(eb, _SCH), pl.ds(col, 128)],
                         msg_v, sem)

    def wait_and_scatter(idx_v, msg_v, sem, col):
        pltpu.make_async_copy(src_hbm.at[pl.ds(0, _SCH)], idx_v, sem).wait()
        pltpu.make_async_copy(msg_hbm.at[pl.ds(0, _SCH), pl.ds(col, 128)],
                              msg_v, sem).wait()
        pltpu.sync_copy(msg_v, acc_sh.at[idx_v], add=True)

    def one_pass(p, carry):
        col = pl.multiple_of(p * 128, 128)
        @pl.when(s < _NS - 1)
        def _():
            pltpu.sync_copy(zeros_hbm.at[pl.ds(0, 624)],
                            acc_sh.at[pl.ds(s * 624, 624)])

        @pl.when(s == _NS - 1)
        def _():
            pltpu.sync_copy(zeros_hbm.at[pl.ds(0, 640)],
                            acc_sh.at[pl.ds(9360, 640)])

        plsc.subcore_barrier()

        start(0, idx_a, msg_a, sem_a, col)

        def pair(j, carry2):
            @pl.when(2 * j + 1 < _NFULL)
            def _():
                start(2 * j + 1, idx_b, msg_b, sem_b, col)

            wait_and_scatter(idx_a, msg_a, sem_a, col)

            @pl.when(2 * j + 2 < _NFULL)
            def _():
                start(2 * j + 2, idx_a, msg_a, sem_a, col)

            @pl.when(2 * j + 1 < _NFULL)
            def _():
                wait_and_scatter(idx_b, msg_b, sem_b, col)

            return carry2

        lax.fori_loop(0, (_NFULL + 1) // 2, pair, 0)

        rb = pl.multiple_of(ebase + _NFULL * _SCH, 8)
        pltpu.sync_copy(src_hbm.at[pl.ds(rb, _REM)], idx_r)
        pltpu.sync_copy(msg_hbm.at[pl.ds(rb, _REM), pl.ds(col, 128)], msg_r)
        pltpu.sync_copy(msg_r, acc_sh.at[idx_r], add=True)
        plsc.subcore_barrier()

        @pl.when(s < _NS - 1)
        def _():
            pltpu.sync_copy(acc_sh.at[pl.ds(s * 624, 624)],
                            out_hbm.at[p, c].at[pl.ds(s * 624, 624)])

        @pl.when(s == _NS - 1)
        def _():
            pltpu.sync_copy(acc_sh.at[pl.ds(9360, 640)],
                            out_hbm.at[p, c].at[pl.ds(9360, 640)])

        plsc.subcore_barrier()
        return carry

    lax.fori_loop(0, _NPASS, one_pass, 0)


def _sc_scatter(msg, src, zeros):
    return pl.kernel(
        _scatter_body,
        out_type=jax.ShapeDtypeStruct((_NPASS, _NC, N, 128), jnp.float32),
        mesh=_sc_mesh(),
        scratch_types=[
            pltpu.VMEM((_SCH,), jnp.int32),
            pltpu.VMEM((_SCH,), jnp.int32),
            pltpu.VMEM((_REM,), jnp.int32),
            pltpu.VMEM((_SCH, 128), jnp.float32),
            pltpu.VMEM((_SCH, 128), jnp.float32),
            pltpu.VMEM((_REM, 128), jnp.float32),
            pltpu.VMEM_SHARED((N, 128), jnp.float32),
            pltpu.SemaphoreType.DMA,
            pltpu.SemaphoreType.DMA,
        ],
    )(msg, src, zeros)


def kernel(x_scalar, x_spherical, rbf, fcut, rsh, edge_index, W1, b1, W2, b2, Wr, br):
    W2p = jnp.pad(W2, ((0, 0), (0, HIDP - HID)))
    b2p = jnp.pad(b2, (0, HIDP - HID))
    Wrp = jnp.pad(Wr, ((0, 0), (0, HIDP - HID)))
    brp = jnp.pad(br, (0, HIDP - HID))
    table = _mlp(x_scalar, x_spherical, W1, b1, W2p, b2p)
    sel2 = jnp.asarray(_SEL2).astype(jnp.bfloat16)
    dst = edge_index[1]
    src = edge_index[0]
    g_tab = _sc_gather(table, dst, 0, E)
    msg = _edge_math(g_tab, rbf, fcut, rsh, Wrp.astype(jnp.bfloat16), brp, sel2,
                     0, E)
    zeros = jnp.zeros((640, 128), jnp.float32)
    parts = _sc_scatter(msg, src, zeros)
    return tuple(_combine(x_scalar, x_spherical, parts))


# speedup vs baseline: 1.0419x; 1.0177x over previous
import functools

import jax
import jax.numpy as jnp
import numpy as np
from jax import lax
from jax.experimental import pallas as pl
from jax.experimental.pallas import tpu as pltpu
from jax.experimental.pallas import tpu_sc as plsc

N = 10000
E = 160000
NODE = 128
NUM_IRR = 224
SPH = 480
NB = 20
HID = NODE + NUM_IRR * 2

_reps = np.array([1] * 128 + [3] * 64 + [5] * 32)
_col_of = np.repeat(np.arange(NUM_IRR), _reps)
_SEL_TAIL = np.zeros((96, 352), dtype=np.float32)
for _j, _c in enumerate(_col_of[128:]):
    _SEL_TAIL[_c - 128, _j] = 1.0
_SEL2 = np.zeros((192, 704), dtype=np.float32)
_SEL2[:96, :352] = _SEL_TAIL
_SEL2[96:, 352:] = _SEL_TAIL

BN = 1000
BE = 2560
HIDP = 640
SPHP = 512


def _mlp_body(x_ref, xsph_ref, w1_ref, b1_ref, w2_ref, b2_ref, o_ref):
    x = x_ref[...]
    h = jnp.dot(x, w1_ref[...], preferred_element_type=jnp.float32,
                precision=jax.lax.Precision.HIGHEST) + b1_ref[...]
    h = h * jax.nn.sigmoid(h)
    so = jnp.dot(h.astype(jnp.bfloat16), w2_ref[...].astype(jnp.bfloat16),
                 preferred_element_type=jnp.float32) + b2_ref[...]
    pad = jnp.zeros((so.shape[0], HIDP - SPH), jnp.float32)
    hi = jnp.concatenate([xsph_ref[...], pad], axis=1)

    def rnd(u):
        return (u + 0x7FFF + ((u >> 16) & 1)) >> 16

    ulo = rnd(jax.lax.bitcast_convert_type(so, jnp.uint32))
    uhi = rnd(jax.lax.bitcast_convert_type(hi, jnp.uint32))
    o_ref[...] = jax.lax.bitcast_convert_type(ulo | (uhi << 16), jnp.float32)


def _edge_body(gtab_ref, rbf_ref, fcut_ref, rsh_ref,
               wr_ref, br_ref, sel_ref, msg_ref):
    fw = jnp.dot(rbf_ref[...].astype(jnp.bfloat16), wr_ref[...],
                 preferred_element_type=jnp.float32) + br_ref[...]
    fw = fw * fcut_ref[...]
    u = jax.lax.bitcast_convert_type(gtab_ref[...], jnp.uint32)
    gso = jax.lax.bitcast_convert_type(u << 16, jnp.float32)
    gsph = jax.lax.bitcast_convert_type(u & jnp.uint32(0xFFFF0000), jnp.float32)
    fo = gso * fw
    ms = fo[:, 2 * NUM_IRR:HID]
    gtails = jnp.concatenate([fo[:, 128:NUM_IRR], fo[:, NUM_IRR + 128:2 * NUM_IRR]],
                             axis=1).astype(jnp.bfloat16)
    tails = jnp.dot(gtails, sel_ref[...], preferred_element_type=jnp.float32)
    rep_gs = jnp.concatenate([fo[:, :128], tails[:, :352]], axis=1)
    rep_ge = jnp.concatenate([fo[:, NUM_IRR:NUM_IRR + 128], tails[:, 352:]],
                             axis=1)
    msph = gsph[:, :SPH] * rep_gs + rsh_ref[...] * rep_ge
    pad = jnp.zeros((msph.shape[0], HIDP - NODE - SPH), jnp.float32)
    msg_ref[...] = jnp.concatenate([ms, msph, pad], axis=1)


TBW = HIDP


def _mlp(x_scalar, x_spherical, W1, b1, W2, b2):
    return pl.pallas_call(
        _mlp_body,
        grid=(N // BN,),
        in_specs=[
            pl.BlockSpec((BN, NODE), lambda i: (i, 0)),
            pl.BlockSpec((BN, SPH), lambda i: (i, 0)),
            pl.BlockSpec((NODE, NODE), lambda i: (0, 0)),
            pl.BlockSpec((NODE,), lambda i: (0,)),
            pl.BlockSpec((NODE, HIDP), lambda i: (0, 0)),
            pl.BlockSpec((HIDP,), lambda i: (0,)),
        ],
        out_specs=pl.BlockSpec((BN, TBW), lambda i: (i, 0)),
        out_shape=jax.ShapeDtypeStruct((N, TBW), jnp.float32),
    )(x_scalar, x_spherical, W1, b1, W2, b2)


def _edge_math(g_tab, rbf, fcut, rsh, Wr, br, sel, ebase, ecount):
    off = ebase // BE
    return pl.pallas_call(
        _edge_body,
        grid=(ecount // BE,),
        in_specs=[
            pl.BlockSpec((BE, TBW), lambda i: (i, 0)),
            pl.BlockSpec((BE, NB), lambda i: (i + off, 0)),
            pl.BlockSpec((BE, 1), lambda i: (i + off, 0)),
            pl.BlockSpec((BE, SPH), lambda i: (i + off, 0)),
            pl.BlockSpec((NB, HIDP), lambda i: (0, 0)),
            pl.BlockSpec((HIDP,), lambda i: (0,)),
            pl.BlockSpec((192, 704), lambda i: (0, 0)),
        ],
        out_specs=pl.BlockSpec((BE, HIDP), lambda i: (i, 0)),
        out_shape=jax.ShapeDtypeStruct((ecount, HIDP), jnp.float32),
    )(g_tab, rbf, fcut, rsh, Wr, br, sel)


def _combine_body(xs_ref, xsph_ref, parts_ref, ns_ref, nsph_ref):
    p = parts_ref[...]
    q = p[:, 0] + p[:, 1]
    ns_ref[...] = xs_ref[...] + q[0]
    sph = jnp.concatenate([q[1], q[2], q[3], q[4]], axis=1)[:, :SPH]
    nsph_ref[...] = xsph_ref[...] + sph


def _combine(x_scalar, x_spherical, parts):
    return pl.pallas_call(
        _combine_body,
        grid=(N // BN,),
        in_specs=[
            pl.BlockSpec((BN, NODE), lambda i: (i, 0)),
            pl.BlockSpec((BN, SPH), lambda i: (i, 0)),
            pl.BlockSpec((_NPASS, _NC, BN, 128), lambda i: (0, 0, i, 0)),
        ],
        out_specs=[
            pl.BlockSpec((BN, NODE), lambda i: (i, 0)),
            pl.BlockSpec((BN, SPH), lambda i: (i, 0)),
        ],
        out_shape=[
            jax.ShapeDtypeStruct((N, NODE), jnp.float32),
            jax.ShapeDtypeStruct((N, SPH), jnp.float32),
        ],
    )(x_scalar, x_spherical, parts)


_NC, _NS = 2, 16
_NW = _NC * _NS
_EPW = E // _NW
_GCH = 40

def _sc_mesh():
    return plsc.VectorSubcoreMesh(core_axis_name="c", subcore_axis_name="s")


def _gather_body(epw, ebase, tab_hbm, dst_hbm, out_tab,
                 idx_all, buf_a, buf_b,
                 gsem_a, gsem_b, wsem_a, wsem_b):
    ngch = epw // _GCH
    wid = lax.axis_index("s") * _NC + lax.axis_index("c")
    base = wid * epw
    pltpu.sync_copy(dst_hbm.at[pl.ds(ebase + base, epw)], idx_all.at[pl.ds(0, epw)])

    def gstart(chunk, buf, gsem):
        off = pl.multiple_of(chunk * _GCH, 8)
        pltpu.async_copy(tab_hbm.at[idx_all.at[pl.ds(off, _GCH)]], buf, gsem)

    def gwait(buf, gsem):
        pltpu.make_async_copy(tab_hbm.at[pl.ds(0, _GCH)], buf, gsem).wait()

    def wstart(chunk, buf, wsem):
        cb = pl.multiple_of(base + chunk * _GCH, 8)
        pltpu.async_copy(buf, out_tab.at[pl.ds(cb, _GCH)], wsem)

    def wwait(buf, wsem):
        pltpu.make_async_copy(buf, out_tab.at[pl.ds(0, _GCH)], wsem).wait()

    gstart(0, buf_a, gsem_a)
    gstart(1, buf_b, gsem_b)

    def pair(j, carry):
        gwait(buf_a, gsem_a)
        wstart(2 * j, buf_a, wsem_a)

        @pl.when(2 * j + 1 < ngch)
        def _():
            gwait(buf_b, gsem_b)
            wstart(2 * j + 1, buf_b, wsem_b)

        @pl.when(2 * j + 2 < ngch)
        def _():
            wwait(buf_a, wsem_a)
            gstart(2 * j + 2, buf_a, gsem_a)

        @pl.when(2 * j + 3 < ngch)
        def _():
            wwait(buf_b, wsem_b)
            gstart(2 * j + 3, buf_b, gsem_b)

        return carry

    lax.fori_loop(0, (ngch + 1) // 2, pair, 0)
    wwait(buf_a, wsem_a)
    wwait(buf_b, wsem_b)


def _sc_gather(table, dst, ebase, ecount):
    return pl.kernel(
        functools.partial(_gather_body, ecount // _NW, ebase),
        out_type=jax.ShapeDtypeStruct((ecount, TBW), jnp.float32),
        mesh=_sc_mesh(),
        scratch_types=[
            pltpu.VMEM((_EPW,), jnp.int32),
            pltpu.VMEM((_GCH, TBW), jnp.float32),
            pltpu.VMEM((_GCH, TBW), jnp.float32),
            pltpu.SemaphoreType.DMA,
            pltpu.SemaphoreType.DMA,
            pltpu.SemaphoreType.DMA,
            pltpu.SemaphoreType.DMA,
        ],
    )(table, dst)


_SCH = 128
_NPASS = HIDP // 128
_NFULL = _EPW // _SCH
_REM = _EPW - _NFULL * _SCH
_EPC = E // _NC


def _scatter_body(msg_hbm, src_hbm, zeros_hbm, out_hbm,
                  idx_a, idx_b, idx_r, msg_a, msg_b, msg_r,
                  acc_sh, sem_a, sem_b):
    c = lax.axis_index("c")
    s = lax.axis_index("s")
    ebase = c * _EPC + s * _EPW

    def start(j, idx_v, msg_v, sem, col):
        eb = pl.multiple_of(ebase + j * _SCH, 8)
        pltpu.async_copy(src_hbm.at[pl.ds(eb, _SCH)], idx_v, sem)
        pltpu.async_copy(msg_hbm.at[pl.ds(eb, _SCH), pl.ds(col, 128)],
                         msg_v, sem)

    def wait_and_scatter(idx_v, msg_v, sem, col):
        pltpu.make_async_copy(src_hbm.at[pl.ds(0, _SCH)], idx_v, sem).wait()
        pltpu.make_async_copy(msg_hbm.at[pl.ds(0, _SCH), pl.ds(col, 128)],
                              msg_v, sem).wait()
        pltpu.sync_copy(msg_v, acc_sh.at[idx_v], add=True)

    def one_pass(p, carry):
        col = pl.multiple_of(p * 128, 128)
        @pl.when(s < _NS - 1)
        def _():
            pltpu.sync_copy(zeros_hbm.at[pl.ds(0, 624)],
                            acc_sh.at[pl.ds(s * 624, 624)])

        @pl.when(s == _NS - 1)
        def _():
            pltpu.sync_copy(zeros_hbm.at[pl.ds(0, 640)],
                            acc_sh.at[pl.ds(9360, 640)])

        plsc.subcore_barrier()

        start(0, idx_a, msg_a, sem_a, col)

        def pair(j, carry2):
            @pl.when(2 * j + 1 < _NFULL)
            def _():
                start(2 * j + 1, idx_b, msg_b, sem_b, col)

            wait_and_scatter(idx_a, msg_a, sem_a, col)

            @pl.when(2 * j + 2 < _NFULL)
            def _():
                start(2 * j + 2, idx_a, msg_a, sem_a, col)

            @pl.when(2 * j + 1 < _NFULL)
            def _():
                wait_and_scatter(idx_b, msg_b, sem_b, col)

            return carry2

        lax.fori_loop(0, (_NFULL + 1) // 2, pair, 0)

        rb = pl.multiple_of(ebase + _NFULL * _SCH, 8)
        pltpu.sync_copy(src_hbm.at[pl.ds(rb, _REM)], idx_r)
        pltpu.sync_copy(msg_hbm.at[pl.ds(rb, _REM), pl.ds(col, 128)], msg_r)
        pltpu.sync_copy(msg_r, acc_sh.at[idx_r], add=True)
        plsc.subcore_barrier()

        @pl.when(s < _NS - 1)
        def _():
            pltpu.sync_copy(acc_sh.at[pl.ds(s * 624, 624)],
                            out_hbm.at[p, c].at[pl.ds(s * 624, 624)])

        @pl.when(s == _NS - 1)
        def _():
            pltpu.sync_copy(acc_sh.at[pl.ds(9360, 640)],
                            out_hbm.at[p, c].at[pl.ds(9360, 640)])

        plsc.subcore_barrier()
        return carry

    lax.fori_loop(0, _NPASS, one_pass, 0)


def _sc_scatter(msg, src, zeros):
    return pl.kernel(
        _scatter_body,
        out_type=jax.ShapeDtypeStruct((_NPASS, _NC, N, 128), jnp.float32),
        mesh=_sc_mesh(),
        scratch_types=[
            pltpu.VMEM((_SCH,), jnp.int32),
            pltpu.VMEM((_SCH,), jnp.int32),
            pltpu.VMEM((_REM,), jnp.int32),
            pltpu.VMEM((_SCH, 128), jnp.float32),
            pltpu.VMEM((_SCH, 128), jnp.float32),
            pltpu.VMEM((_REM, 128), jnp.float32),
            pltpu.VMEM_SHARED((N, 128), jnp.float32),
            pltpu.SemaphoreType.DMA,
            pltpu.SemaphoreType.DMA,
        ],
    )(msg, src, zeros)


def kernel(x_scalar, x_spherical, rbf, fcut, rsh, edge_index, W1, b1, W2, b2, Wr, br):
    W2p = jnp.pad(W2, ((0, 0), (0, HIDP - HID)))
    b2p = jnp.pad(b2, (0, HIDP - HID))
    Wrp = jnp.pad(Wr, ((0, 0), (0, HIDP - HID)))
    brp = jnp.pad(br, (0, HIDP - HID))
    table = _mlp(x_scalar, x_spherical, W1, b1, W2p, b2p)
    sel2 = jnp.asarray(_SEL2).astype(jnp.bfloat16)
    dst = edge_index[1]
    src = edge_index[0]
    g_tab = _sc_gather(table, dst, 0, E)
    msg = _edge_math(g_tab, rbf, fcut, rsh, Wrp.astype(jnp.bfloat16), brp, sel2,
                     0, E)
    zeros = jnp.zeros((640, 128), jnp.float32)
    parts = _sc_scatter(msg, src, zeros)
    return tuple(_combine(x_scalar, x_spherical, parts))


# speedup vs baseline: 1.0420x; 1.0001x over previous
import functools

import jax
import jax.numpy as jnp
import numpy as np
from jax import lax
from jax.experimental import pallas as pl
from jax.experimental.pallas import tpu as pltpu
from jax.experimental.pallas import tpu_sc as plsc

N = 10000
E = 160000
NODE = 128
NUM_IRR = 224
SPH = 480
NB = 20
HID = NODE + NUM_IRR * 2

_reps = np.array([1] * 128 + [3] * 64 + [5] * 32)
_col_of = np.repeat(np.arange(NUM_IRR), _reps)
_SEL_TAIL = np.zeros((96, 352), dtype=np.float32)
for _j, _c in enumerate(_col_of[128:]):
    _SEL_TAIL[_c - 128, _j] = 1.0
_SEL2 = np.zeros((192, 704), dtype=np.float32)
_SEL2[:96, :352] = _SEL_TAIL
_SEL2[96:, 352:] = _SEL_TAIL

BN = 1000
BE = 1280
HIDP = 640
SPHP = 512


def _mlp_body(x_ref, xsph_ref, w1_ref, b1_ref, w2_ref, b2_ref, o_ref):
    x = x_ref[...]
    h = jnp.dot(x, w1_ref[...], preferred_element_type=jnp.float32,
                precision=jax.lax.Precision.HIGHEST) + b1_ref[...]
    h = h * jax.nn.sigmoid(h)
    so = jnp.dot(h.astype(jnp.bfloat16), w2_ref[...].astype(jnp.bfloat16),
                 preferred_element_type=jnp.float32) + b2_ref[...]
    pad = jnp.zeros((so.shape[0], HIDP - SPH), jnp.float32)
    hi = jnp.concatenate([xsph_ref[...], pad], axis=1)

    def rnd(u):
        return (u + 0x7FFF + ((u >> 16) & 1)) >> 16

    ulo = rnd(jax.lax.bitcast_convert_type(so, jnp.uint32))
    uhi = rnd(jax.lax.bitcast_convert_type(hi, jnp.uint32))
    o_ref[...] = jax.lax.bitcast_convert_type(ulo | (uhi << 16), jnp.float32)


def _edge_body(gtab_ref, rbf_ref, fcut_ref, rsh_ref,
               wr_ref, br_ref, sel_ref, msg_ref):
    fw = jnp.dot(rbf_ref[...].astype(jnp.bfloat16), wr_ref[...],
                 preferred_element_type=jnp.float32) + br_ref[...]
    fw = fw * fcut_ref[...]
    u = jax.lax.bitcast_convert_type(gtab_ref[...], jnp.uint32)
    gso = jax.lax.bitcast_convert_type(u << 16, jnp.float32)
    gsph = jax.lax.bitcast_convert_type(u & jnp.uint32(0xFFFF0000), jnp.float32)
    fo = gso * fw
    ms = fo[:, 2 * NUM_IRR:HID]
    gtails = jnp.concatenate([fo[:, 128:NUM_IRR], fo[:, NUM_IRR + 128:2 * NUM_IRR]],
                             axis=1).astype(jnp.bfloat16)
    tails = jnp.dot(gtails, sel_ref[...], preferred_element_type=jnp.float32)
    rep_gs = jnp.concatenate([fo[:, :128], tails[:, :352]], axis=1)
    rep_ge = jnp.concatenate([fo[:, NUM_IRR:NUM_IRR + 128], tails[:, 352:]],
                             axis=1)
    msph = gsph[:, :SPH] * rep_gs + rsh_ref[...] * rep_ge
    pad = jnp.zeros((msph.shape[0], HIDP - NODE - SPH), jnp.float32)
    msg_ref[...] = jnp.concatenate([ms, msph, pad], axis=1)


TBW = HIDP


def _mlp(x_scalar, x_spherical, W1, b1, W2, b2):
    return pl.pallas_call(
        _mlp_body,
        grid=(N // BN,),
        in_specs=[
            pl.BlockSpec((BN, NODE), lambda i: (i, 0)),
            pl.BlockSpec((BN, SPH), lambda i: (i, 0)),
            pl.BlockSpec((NODE, NODE), lambda i: (0, 0)),
            pl.BlockSpec((NODE,), lambda i: (0,)),
            pl.BlockSpec((NODE, HIDP), lambda i: (0, 0)),
            pl.BlockSpec((HIDP,), lambda i: (0,)),
        ],
        out_specs=pl.BlockSpec((BN, TBW), lambda i: (i, 0)),
        out_shape=jax.ShapeDtypeStruct((N, TBW), jnp.float32),
    )(x_scalar, x_spherical, W1, b1, W2, b2)


def _edge_math(g_tab, rbf, fcut, rsh, Wr, br, sel, ebase, ecount):
    off = ebase // BE
    return pl.pallas_call(
        _edge_body,
        grid=(ecount // BE,),
        in_specs=[
            pl.BlockSpec((BE, TBW), lambda i: (i, 0)),
            pl.BlockSpec((BE, NB), lambda i: (i + off, 0)),
            pl.BlockSpec((BE, 1), lambda i: (i + off, 0)),
            pl.BlockSpec((BE, SPH), lambda i: (i + off, 0)),
            pl.BlockSpec((NB, HIDP), lambda i: (0, 0)),
            pl.BlockSpec((HIDP,), lambda i: (0,)),
            pl.BlockSpec((192, 704), lambda i: (0, 0)),
        ],
        out_specs=pl.BlockSpec((BE, HIDP), lambda i: (i, 0)),
        out_shape=jax.ShapeDtypeStruct((ecount, HIDP), jnp.float32),
    )(g_tab, rbf, fcut, rsh, Wr, br, sel)


def _combine_body(xs_ref, xsph_ref, parts_ref, ns_ref, nsph_ref):
    p = parts_ref[...]
    q = p[:, 0] + p[:, 1]
    ns_ref[...] = xs_ref[...] + q[0]
    sph = jnp.concatenate([q[1], q[2], q[3], q[4]], axis=1)[:, :SPH]
    nsph_ref[...] = xsph_ref[...] + sph


def _combine(x_scalar, x_spherical, parts):
    return pl.pallas_call(
        _combine_body,
        grid=(N // BN,),
        in_specs=[
            pl.BlockSpec((BN, NODE), lambda i: (i, 0)),
            pl.BlockSpec((BN, SPH), lambda i: (i, 0)),
            pl.BlockSpec((_NPASS, _NC, BN, 128), lambda i: (0, 0, i, 0)),
        ],
        out_specs=[
            pl.BlockSpec((BN, NODE), lambda i: (i, 0)),
            pl.BlockSpec((BN, SPH), lambda i: (i, 0)),
        ],
        out_shape=[
            jax.ShapeDtypeStruct((N, NODE), jnp.float32),
            jax.ShapeDtypeStruct((N, SPH), jnp.float32),
        ],
    )(x_scalar, x_spherical, parts)


_NC, _NS = 2, 16
_NW = _NC * _NS
_EPW = E // _NW
_GCH = 40

def _sc_mesh():
    return plsc.VectorSubcoreMesh(core_axis_name="c", subcore_axis_name="s")


def _gather_body(epw, ebase, tab_hbm, dst_hbm, out_tab,
                 idx_all, buf_a, buf_b,
                 gsem_a, gsem_b, wsem_a, wsem_b):
    ngch = epw // _GCH
    wid = lax.axis_index("s") * _NC + lax.axis_index("c")
    base = wid * epw
    pltpu.sync_copy(dst_hbm.at[pl.ds(ebase + base, epw)], idx_all.at[pl.ds(0, epw)])

    def gstart(chunk, buf, gsem):
        off = pl.multiple_of(chunk * _GCH, 8)
        pltpu.async_copy(tab_hbm.at[idx_all.at[pl.ds(off, _GCH)]], buf, gsem)

    def gwait(buf, gsem):
        pltpu.make_async_copy(tab_hbm.at[pl.ds(0, _GCH)], buf, gsem).wait()

    def wstart(chunk, buf, wsem):
        cb = pl.multiple_of(base + chunk * _GCH, 8)
        pltpu.async_copy(buf, out_tab.at[pl.ds(cb, _GCH)], wsem)

    def wwait(buf, wsem):
        pltpu.make_async_copy(buf, out_tab.at[pl.ds(0, _GCH)], wsem).wait()

    gstart(0, buf_a, gsem_a)
    gstart(1, buf_b, gsem_b)

    def pair(j, carry):
        gwait(buf_a, gsem_a)
        wstart(2 * j, buf_a, wsem_a)

        @pl.when(2 * j + 1 < ngch)
        def _():
            gwait(buf_b, gsem_b)
            wstart(2 * j + 1, buf_b, wsem_b)

        @pl.when(2 * j + 2 < ngch)
        def _():
            wwait(buf_a, wsem_a)
            gstart(2 * j + 2, buf_a, gsem_a)

        @pl.when(2 * j + 3 < ngch)
        def _():
            wwait(buf_b, wsem_b)
            gstart(2 * j + 3, buf_b, gsem_b)

        return carry

    lax.fori_loop(0, (ngch + 1) // 2, pair, 0)
    wwait(buf_a, wsem_a)
    wwait(buf_b, wsem_b)


def _sc_gather(table, dst, ebase, ecount):
    return pl.kernel(
        functools.partial(_gather_body, ecount // _NW, ebase),
        out_type=jax.ShapeDtypeStruct((ecount, TBW), jnp.float32),
        mesh=_sc_mesh(),
        scratch_types=[
            pltpu.VMEM((_EPW,), jnp.int32),
            pltpu.VMEM((_GCH, TBW), jnp.float32),
            pltpu.VMEM((_GCH, TBW), jnp.float32),
            pltpu.SemaphoreType.DMA,
            pltpu.SemaphoreType.DMA,
            pltpu.SemaphoreType.DMA,
            pltpu.SemaphoreType.DMA,
        ],
    )(table, dst)


_SCH = 128
_NPASS = HIDP // 128
_NFULL = _EPW // _SCH
_REM = _EPW - _NFULL * _SCH
_EPC = E // _NC


def _scatter_body(msg_hbm, src_hbm, zeros_hbm, out_hbm,
                  idx_a, idx_b, idx_r, msg_a, msg_b, msg_r,
                  acc_sh, sem_a, sem_b):
    c = lax.axis_index("c")
    s = lax.axis_index("s")
    ebase = c * _EPC + s * _EPW

    def start(j, idx_v, msg_v, sem, col):
        eb = pl.multiple_of(ebase + j * _SCH, 8)
        pltpu.async_copy(src_hbm.at[pl.ds(eb, _SCH)], idx_v, sem)
        pltpu.async_copy(msg_hbm.at[pl.ds(eb, _SCH), pl.ds(col, 128)],
                         msg_v, sem)

    def wait_and_scatter(idx_v, msg_v, sem, col):
        pltpu.make_async_copy(src_hbm.at[pl.ds(0, _SCH)], idx_v, sem).wait()
        pltpu.make_async_copy(msg_hbm.at[pl.ds(0, _SCH), pl.ds(col, 128)],
                              msg_v, sem).wait()
        pltpu.sync_copy(msg_v, acc_sh.at[idx_v], add=True)

    def one_pass(p, carry):
        col = pl.multiple_of(p * 128, 128)
        @pl.when(s < _NS - 1)
        def _():
            pltpu.sync_copy(zeros_hbm.at[pl.ds(0, 624)],
                            acc_sh.at[pl.ds(s * 624, 624)])

        @pl.when(s == _NS - 1)
        def _():
            pltpu.sync_copy(zeros_hbm.at[pl.ds(0, 640)],
                            acc_sh.at[pl.ds(9360, 640)])

        plsc.subcore_barrier()

        start(0, idx_a, msg_a, sem_a, col)

        def pair(j, carry2):
            @pl.when(2 * j + 1 < _NFULL)
            def _():
                start(2 * j + 1, idx_b, msg_b, sem_b, col)

            wait_and_scatter(idx_a, msg_a, sem_a, col)

            @pl.when(2 * j + 2 < _NFULL)
            def _():
                start(2 * j + 2, idx_a, msg_a, sem_a, col)

            @pl.when(2 * j + 1 < _NFULL)
            def _():
                wait_and_scatter(idx_b, msg_b, sem_b, col)

            return carry2

        lax.fori_loop(0, (_NFULL + 1) // 2, pair, 0)

        rb = pl.multiple_of(ebase + _NFULL * _SCH, 8)
        pltpu.sync_copy(src_hbm.at[pl.ds(rb, _REM)], idx_r)
        pltpu.sync_copy(msg_hbm.at[pl.ds(rb, _REM), pl.ds(col, 128)], msg_r)
        pltpu.sync_copy(msg_r, acc_sh.at[idx_r], add=True)
        plsc.subcore_barrier()

        @pl.when(s < _NS - 1)
        def _():
            pltpu.sync_copy(acc_sh.at[pl.ds(s * 624, 624)],
                            out_hbm.at[p, c].at[pl.ds(s * 624, 624)])

        @pl.when(s == _NS - 1)
        def _():
            pltpu.sync_copy(acc_sh.at[pl.ds(9360, 640)],
                            out_hbm.at[p, c].at[pl.ds(9360, 640)])

        plsc.subcore_barrier()
        return carry

    lax.fori_loop(0, _NPASS, one_pass, 0)


def _sc_scatter(msg, src, zeros):
    return pl.kernel(
        _scatter_body,
        out_type=jax.ShapeDtypeStruct((_NPASS, _NC, N, 128), jnp.float32),
        mesh=_sc_mesh(),
        scratch_types=[
            pltpu.VMEM((_SCH,), jnp.int32),
            pltpu.VMEM((_SCH,), jnp.int32),
            pltpu.VMEM((_REM,), jnp.int32),
            pltpu.VMEM((_SCH, 128), jnp.float32),
            pltpu.VMEM((_SCH, 128), jnp.float32),
            pltpu.VMEM((_REM, 128), jnp.float32),
            pltpu.VMEM_SHARED((N, 128), jnp.float32),
            pltpu.SemaphoreType.DMA,
            pltpu.SemaphoreType.DMA,
        ],
    )(msg, src, zeros)


def kernel(x_scalar, x_spherical, rbf, fcut, rsh, edge_index, W1, b1, W2, b2, Wr, br):
    W2p = jnp.pad(W2, ((0, 0), (0, HIDP - HID)))
    b2p = jnp.pad(b2, (0, HIDP - HID))
    Wrp = jnp.pad(Wr, ((0, 0), (0, HIDP - HID)))
    brp = jnp.pad(br, (0, HIDP - HID))
    table = _mlp(x_scalar, x_spherical, W1, b1, W2p, b2p)
    sel2 = jnp.asarray(_SEL2).astype(jnp.bfloat16)
    dst = edge_index[1]
    src = edge_index[0]
    g_tab = _sc_gather(table, dst, 0, E)
    msg = _edge_math(g_tab, rbf, fcut, rsh, Wrp.astype(jnp.bfloat16), brp, sel2,
                     0, E)
    zeros = jnp.zeros((640, 128), jnp.float32)
    parts = _sc_scatter(msg, src, zeros)
    return tuple(_combine(x_scalar, x_spherical, parts))


# speedup vs baseline: 1.0429x; 1.0009x over previous
import functools

import jax
import jax.numpy as jnp
import numpy as np
from jax import lax
from jax.experimental import pallas as pl
from jax.experimental.pallas import tpu as pltpu
from jax.experimental.pallas import tpu_sc as plsc

N = 10000
E = 160000
NODE = 128
NUM_IRR = 224
SPH = 480
NB = 20
HID = NODE + NUM_IRR * 2

_reps = np.array([1] * 128 + [3] * 64 + [5] * 32)
_col_of = np.repeat(np.arange(NUM_IRR), _reps)
_SEL_TAIL = np.zeros((96, 352), dtype=np.float32)
for _j, _c in enumerate(_col_of[128:]):
    _SEL_TAIL[_c - 128, _j] = 1.0
_SEL2 = np.zeros((192, 704), dtype=np.float32)
_SEL2[:96, :352] = _SEL_TAIL
_SEL2[96:, 352:] = _SEL_TAIL

BN = 1000
BE = 1280
HIDP = 640
SPHP = 512


def _mlp_body(x_ref, xsph_ref, w1_ref, b1_ref, w2_ref, b2_ref, o_ref):
    x = x_ref[...]
    h = jnp.dot(x, w1_ref[...], preferred_element_type=jnp.float32,
                precision=jax.lax.Precision.HIGHEST) + b1_ref[...]
    h = h * jax.nn.sigmoid(h)
    so = jnp.dot(h.astype(jnp.bfloat16), w2_ref[...].astype(jnp.bfloat16),
                 preferred_element_type=jnp.float32) + b2_ref[...]
    pad = jnp.zeros((so.shape[0], HIDP - SPH), jnp.float32)
    hi = jnp.concatenate([xsph_ref[...], pad], axis=1)

    def rnd(u):
        return (u + 0x7FFF + ((u >> 16) & 1)) >> 16

    ulo = rnd(jax.lax.bitcast_convert_type(so, jnp.uint32))
    uhi = rnd(jax.lax.bitcast_convert_type(hi, jnp.uint32))
    o_ref[...] = jax.lax.bitcast_convert_type(ulo | (uhi << 16), jnp.float32)


def _edge_body(gtab_ref, rbf_ref, fcut_ref, rsh_ref,
               wr_ref, br_ref, sel_ref, msg_ref):
    fw = jnp.dot(rbf_ref[...].astype(jnp.bfloat16), wr_ref[...],
                 preferred_element_type=jnp.float32) + br_ref[...]
    fw = fw * fcut_ref[...]
    u = jax.lax.bitcast_convert_type(gtab_ref[...], jnp.uint32)
    gso = jax.lax.bitcast_convert_type(u << 16, jnp.float32)
    gsph = jax.lax.bitcast_convert_type(u & jnp.uint32(0xFFFF0000), jnp.float32)
    fo = gso * fw
    ms = fo[:, 2 * NUM_IRR:HID]
    gtails = jnp.concatenate([fo[:, 128:NUM_IRR], fo[:, NUM_IRR + 128:2 * NUM_IRR]],
                             axis=1).astype(jnp.bfloat16)
    tails = jnp.dot(gtails, sel_ref[...], preferred_element_type=jnp.float32)
    rep_gs = jnp.concatenate([fo[:, :128], tails[:, :352]], axis=1)
    rep_ge = jnp.concatenate([fo[:, NUM_IRR:NUM_IRR + 128], tails[:, 352:]],
                             axis=1)
    msph = gsph[:, :SPH] * rep_gs + rsh_ref[...] * rep_ge
    pad = jnp.zeros((msph.shape[0], HIDP - NODE - SPH), jnp.float32)
    msg_ref[...] = jnp.concatenate([ms, msph, pad], axis=1)


TBW = HIDP


def _mlp(x_scalar, x_spherical, W1, b1, W2, b2):
    return pl.pallas_call(
        _mlp_body,
        grid=(N // BN,),
        in_specs=[
            pl.BlockSpec((BN, NODE), lambda i: (i, 0)),
            pl.BlockSpec((BN, SPH), lambda i: (i, 0)),
            pl.BlockSpec((NODE, NODE), lambda i: (0, 0)),
            pl.BlockSpec((NODE,), lambda i: (0,)),
            pl.BlockSpec((NODE, HIDP), lambda i: (0, 0)),
            pl.BlockSpec((HIDP,), lambda i: (0,)),
        ],
        out_specs=pl.BlockSpec((BN, TBW), lambda i: (i, 0)),
        out_shape=jax.ShapeDtypeStruct((N, TBW), jnp.float32),
    )(x_scalar, x_spherical, W1, b1, W2, b2)


def _edge_math(g_tab, rbf, fcut, rsh, Wr, br, sel, ebase, ecount):
    off = ebase // BE
    return pl.pallas_call(
        _edge_body,
        grid=(ecount // BE,),
        in_specs=[
            pl.BlockSpec((BE, TBW), lambda i: (i, 0)),
            pl.BlockSpec((BE, NB), lambda i: (i + off, 0)),
            pl.BlockSpec((BE, 1), lambda i: (i + off, 0)),
            pl.BlockSpec((BE, SPH), lambda i: (i + off, 0)),
            pl.BlockSpec((NB, HIDP), lambda i: (0, 0)),
            pl.BlockSpec((HIDP,), lambda i: (0,)),
            pl.BlockSpec((192, 704), lambda i: (0, 0)),
        ],
        out_specs=pl.BlockSpec((BE, HIDP), lambda i: (i, 0)),
        out_shape=jax.ShapeDtypeStruct((ecount, HIDP), jnp.float32),
    )(g_tab, rbf, fcut, rsh, Wr, br, sel)


def _combine_body(xs_ref, xsph_ref, parts_ref, ns_ref, nsph_ref):
    p = parts_ref[...]
    q = p[:, 0] + p[:, 1]
    ns_ref[...] = xs_ref[...] + q[0]
    sph = jnp.concatenate([q[1], q[2], q[3], q[4]], axis=1)[:, :SPH]
    nsph_ref[...] = xsph_ref[...] + sph


def _combine(x_scalar, x_spherical, parts):
    return pl.pallas_call(
        _combine_body,
        grid=(N // BN,),
        in_specs=[
            pl.BlockSpec((BN, NODE), lambda i: (i, 0)),
            pl.BlockSpec((BN, SPH), lambda i: (i, 0)),
            pl.BlockSpec((_NPASS, _NC, BN, 128), lambda i: (0, 0, i, 0)),
        ],
        out_specs=[
            pl.BlockSpec((BN, NODE), lambda i: (i, 0)),
            pl.BlockSpec((BN, SPH), lambda i: (i, 0)),
        ],
        out_shape=[
            jax.ShapeDtypeStruct((N, NODE), jnp.float32),
            jax.ShapeDtypeStruct((N, SPH), jnp.float32),
        ],
    )(x_scalar, x_spherical, parts)


_NC, _NS = 2, 16
_NW = _NC * _NS
_EPW = E // _NW
_GCH = 40

def _sc_mesh():
    return plsc.VectorSubcoreMesh(core_axis_name="c", subcore_axis_name="s")


def _gather_body(epw, ebase, tab_hbm, dst_hbm, out_tab,
                 idx_all, buf_a, buf_b, buf_c,
                 gsem_a, gsem_b, gsem_c, wsem_a, wsem_b, wsem_c):
    ngch = epw // _GCH
    wid = lax.axis_index("s") * _NC + lax.axis_index("c")
    base = wid * epw
    pltpu.sync_copy(dst_hbm.at[pl.ds(ebase + base, epw)], idx_all.at[pl.ds(0, epw)])

    def gstart(chunk, buf, gsem):
        off = pl.multiple_of(chunk * _GCH, 8)
        pltpu.async_copy(tab_hbm.at[idx_all.at[pl.ds(off, _GCH)]], buf, gsem)

    def gwait(buf, gsem):
        pltpu.make_async_copy(tab_hbm.at[pl.ds(0, _GCH)], buf, gsem).wait()

    def wstart(chunk, buf, wsem):
        cb = pl.multiple_of(base + chunk * _GCH, 8)
        pltpu.async_copy(buf, out_tab.at[pl.ds(cb, _GCH)], wsem)

    def wwait(buf, wsem):
        pltpu.make_async_copy(buf, out_tab.at[pl.ds(0, _GCH)], wsem).wait()

    gstart(0, buf_a, gsem_a)
    gstart(1, buf_b, gsem_b)
    gstart(2, buf_c, gsem_c)

    def tri(j, carry):
        gwait(buf_a, gsem_a)
        wstart(3 * j, buf_a, wsem_a)

        @pl.when(3 * j + 1 < ngch)
        def _():
            gwait(buf_b, gsem_b)
            wstart(3 * j + 1, buf_b, wsem_b)

        @pl.when(3 * j + 3 < ngch)
        def _():
            wwait(buf_a, wsem_a)
            gstart(3 * j + 3, buf_a, gsem_a)

        @pl.when(3 * j + 2 < ngch)
        def _():
            gwait(buf_c, gsem_c)
            wstart(3 * j + 2, buf_c, wsem_c)

        @pl.when(3 * j + 4 < ngch)
        def _():
            wwait(buf_b, wsem_b)
            gstart(3 * j + 4, buf_b, gsem_b)

        @pl.when(3 * j + 5 < ngch)
        def _():
            wwait(buf_c, wsem_c)
            gstart(3 * j + 5, buf_c, gsem_c)

        return carry

    lax.fori_loop(0, (ngch + 2) // 3, tri, 0)
    wwait(buf_a, wsem_a)
    wwait(buf_b, wsem_b)
    wwait(buf_c, wsem_c)


def _sc_gather(table, dst, ebase, ecount):
    return pl.kernel(
        functools.partial(_gather_body, ecount // _NW, ebase),
        out_type=jax.ShapeDtypeStruct((ecount, TBW), jnp.float32),
        mesh=_sc_mesh(),
        scratch_types=[
            pltpu.VMEM((_EPW,), jnp.int32),
            pltpu.VMEM((_GCH, TBW), jnp.float32),
            pltpu.VMEM((_GCH, TBW), jnp.float32),
            pltpu.VMEM((_GCH, TBW), jnp.float32),
            pltpu.SemaphoreType.DMA,
            pltpu.SemaphoreType.DMA,
            pltpu.SemaphoreType.DMA,
            pltpu.SemaphoreType.DMA,
            pltpu.SemaphoreType.DMA,
            pltpu.SemaphoreType.DMA,
        ],
    )(table, dst)


_SCH = 128
_NPASS = HIDP // 128
_NFULL = _EPW // _SCH
_REM = _EPW - _NFULL * _SCH
_EPC = E // _NC


def _scatter_body(msg_hbm, src_hbm, zeros_hbm, out_hbm,
                  idx_a, idx_b, idx_r, msg_a, msg_b, msg_r,
                  acc_sh, sem_a, sem_b):
    c = lax.axis_index("c")
    s = lax.axis_index("s")
    ebase = c * _EPC + s * _EPW

    def start(j, idx_v, msg_v, sem, col):
        eb = pl.multiple_of(ebase + j * _SCH, 8)
        pltpu.async_copy(src_hbm.at[pl.ds(eb, _SCH)], idx_v, sem)
        pltpu.async_copy(msg_hbm.at[pl.ds(eb, _SCH), pl.ds(col, 128)],
                         msg_v, sem)

    def wait_and_scatter(idx_v, msg_v, sem, col):
        pltpu.make_async_copy(src_hbm.at[pl.ds(0, _SCH)], idx_v, sem).wait()
        pltpu.make_async_copy(msg_hbm.at[pl.ds(0, _SCH), pl.ds(col, 128)],
                              msg_v, sem).wait()
        pltpu.sync_copy(msg_v, acc_sh.at[idx_v], add=True)

    def one_pass(p, carry):
        col = pl.multiple_of(p * 128, 128)
        @pl.when(s < _NS - 1)
        def _():
            pltpu.sync_copy(zeros_hbm.at[pl.ds(0, 624)],
                            acc_sh.at[pl.ds(s * 624, 624)])

        @pl.when(s == _NS - 1)
        def _():
            pltpu.sync_copy(zeros_hbm.at[pl.ds(0, 640)],
                            acc_sh.at[pl.ds(9360, 640)])

        plsc.subcore_barrier()

        start(0, idx_a, msg_a, sem_a, col)

        def pair(j, carry2):
            @pl.when(2 * j + 1 < _NFULL)
            def _():
                start(2 * j + 1, idx_b, msg_b, sem_b, col)

            wait_and_scatter(idx_a, msg_a, sem_a, col)

            @pl.when(2 * j + 2 < _NFULL)
            def _():
                start(2 * j + 2, idx_a, msg_a, sem_a, col)

            @pl.when(2 * j + 1 < _NFULL)
            def _():
                wait_and_scatter(idx_b, msg_b, sem_b, col)

            return carry2

        lax.fori_loop(0, (_NFULL + 1) // 2, pair, 0)

        rb = pl.multiple_of(ebase + _NFULL * _SCH, 8)
        pltpu.sync_copy(src_hbm.at[pl.ds(rb, _REM)], idx_r)
        pltpu.sync_copy(msg_hbm.at[pl.ds(rb, _REM), pl.ds(col, 128)], msg_r)
        pltpu.sync_copy(msg_r, acc_sh.at[idx_r], add=True)
        plsc.subcore_barrier()

        @pl.when(s < _NS - 1)
        def _():
            pltpu.sync_copy(acc_sh.at[pl.ds(s * 624, 624)],
                            out_hbm.at[p, c].at[pl.ds(s * 624, 624)])

        @pl.when(s == _NS - 1)
        def _():
            pltpu.sync_copy(acc_sh.at[pl.ds(9360, 640)],
                            out_hbm.at[p, c].at[pl.ds(9360, 640)])

        plsc.subcore_barrier()
        return carry

    lax.fori_loop(0, _NPASS, one_pass, 0)


def _sc_scatter(msg, src, zeros):
    return pl.kernel(
        _scatter_body,
        out_type=jax.ShapeDtypeStruct((_NPASS, _NC, N, 128), jnp.float32),
        mesh=_sc_mesh(),
        scratch_types=[
            pltpu.VMEM((_SCH,), jnp.int32),
            pltpu.VMEM((_SCH,), jnp.int32),
            pltpu.VMEM((_REM,), jnp.int32),
            pltpu.VMEM((_SCH, 128), jnp.float32),
            pltpu.VMEM((_SCH, 128), jnp.float32),
            pltpu.VMEM((_REM, 128), jnp.float32),
            pltpu.VMEM_SHARED((N, 128), jnp.float32),
            pltpu.SemaphoreType.DMA,
            pltpu.SemaphoreType.DMA,
        ],
    )(msg, src, zeros)


def kernel(x_scalar, x_spherical, rbf, fcut, rsh, edge_index, W1, b1, W2, b2, Wr, br):
    W2p = jnp.pad(W2, ((0, 0), (0, HIDP - HID)))
    b2p = jnp.pad(b2, (0, HIDP - HID))
    Wrp = jnp.pad(Wr, ((0, 0), (0, HIDP - HID)))
    brp = jnp.pad(br, (0, HIDP - HID))
    table = _mlp(x_scalar, x_spherical, W1, b1, W2p, b2p)
    sel2 = jnp.asarray(_SEL2).astype(jnp.bfloat16)
    dst = edge_index[1]
    src = edge_index[0]
    g_tab = _sc_gather(table, dst, 0, E)
    msg = _edge_math(g_tab, rbf, fcut, rsh, Wrp.astype(jnp.bfloat16), brp, sel2,
                     0, E)
    zeros = jnp.zeros((640, 128), jnp.float32)
    parts = _sc_scatter(msg, src, zeros)
    return tuple(_combine(x_scalar, x_spherical, parts))
